# Initial kernel scaffold; baseline (speedup 1.0000x reference)
import jax
import jax.numpy as jnp
from jax.experimental import pallas as pl


def kernel(item_emb, inter_rows, inter_cols):
    raise NotImplementedError("write your pallas kernel here")


# speedup vs baseline: 12.8858x; 12.8858x over previous
import functools

import jax
import jax.numpy as jnp
from jax import lax
from jax.experimental import pallas as pl
from jax.experimental.pallas import tpu as pltpu
from jax.experimental.pallas import tpu_sc as plsc

N = 25000
HR = 25088
NNZ = 400000
NNZ_PAD = 401408
EPT = NNZ_PAD // 16
CK = 128
G = 7
NG = EPT // (G * CK)
RPT = HR // 16
RC = 32
NRC = RPT // RC

_MAGIC = 0x5F3759DF


def _tanh(x):
    t = jnp.exp(2.0 * x)
    return 1.0 - 2.0 / (t + 1.0)


def _rsqrt_pos(x):
    xm = jnp.maximum(x, 1.0)
    ib = lax.bitcast_convert_type(xm, jnp.int32)
    ih = jnp.int32(_MAGIC) - lax.shift_right_logical(
        ib, jnp.full((16,), 1, jnp.int32))
    y = lax.bitcast_convert_type(ih, jnp.float32)
    y = y * (1.5 - 0.5 * xm * y * y)
    y = y * (1.5 - 0.5 * xm * y * y)
    y = y * (1.5 - 0.5 * xm * y * y)
    return jnp.where(x > 0.0, y, 0.0)


_mesh = plsc.VectorSubcoreMesh(core_axis_name="c", subcore_axis_name="s")


@functools.partial(
    pl.kernel,
    out_type=[jax.ShapeDtypeStruct((2, HR, 64), jnp.float32)
              for _ in range(6)],
    mesh=_mesh,
    compiler_params=pltpu.CompilerParams(use_tc_tiling_on_sc=False),
    scratch_types=[
        pltpu.VMEM_SHARED((HR, 64), jnp.float32),
        pltpu.VMEM_SHARED((2, HR), jnp.float32),
        pltpu.VMEM((2, G, CK), jnp.int32),
        pltpu.VMEM((CK, 64), jnp.float32),
        pltpu.VMEM((CK, 64), jnp.float32),
        pltpu.VMEM((RC, 64), jnp.float32),
        pltpu.VMEM((RC, 64), jnp.float32),
        pltpu.VMEM((RPT,), jnp.float32),
        pltpu.VMEM((CK,), jnp.float32),
        pltpu.SemaphoreType.DMA,
        pltpu.SemaphoreType.DMA,
    ],
)
def _egcf(e_ref, eidx_ref,
          users_ref, items_ref, u0_ref, s1_ref, s2_ref, s3_ref,
          acc, dudi, eib, gb0, gb1, rbuf, sbuf, ddb, onesb, sem0, sem1):
    k = lax.axis_index("c")
    s = lax.axis_index("s")
    gbufs = (gb0, gb1)
    sems = (sem0, sem1)

    def fill(ref, words, value):
        def body(i, carry):
            ref[pl.ds(i * 16, 16)] = jnp.full((16,), value, jnp.float32)
            return carry
        lax.fori_loop(0, words // 16, body, None)

    def zero_acc():
        def zr(r, carry):
            for j in range(4):
                rbuf[r, pl.ds(j * 16, 16)] = jnp.zeros((16,), jnp.float32)
            return carry
        lax.fori_loop(0, RC, zr, None)
        def zc(c, carry):
            pltpu.sync_copy(rbuf, acc.at[pl.ds(s * RPT + c * RC, RC)])
            return carry
        lax.fori_loop(0, NRC, zc, None)

    fill(ddb, RPT, 0.0)
    for h in range(2):
        pltpu.sync_copy(ddb, dudi.at[h, pl.ds(s * RPT, RPT)])
    plsc.subcore_barrier()

    fill(onesb, CK, 1.0)
    def deg_group(g, carry):
        pltpu.sync_copy(eidx_ref.at[s, g], eib)
        for j in range(G):
            pltpu.sync_copy(onesb, dudi.at[0].at[eib.at[0, j]], add=True)
            pltpu.sync_copy(onesb, dudi.at[1].at[eib.at[1, j]], add=True)
        return carry
    lax.fori_loop(0, NG, deg_group, None)
    plsc.subcore_barrier()

    for h in range(2):
        pltpu.sync_copy(dudi.at[h, pl.ds(s * RPT, RPT)], ddb)
        def rs_body(i, carry):
            sl = pl.ds(i * 16, 16)
            ddb[sl] = _rsqrt_pos(ddb[sl])
            return carry
        lax.fori_loop(0, RPT // 16, rs_body, None)
        pltpu.sync_copy(ddb, dudi.at[h, pl.ds(s * RPT, RPT)])
    plsc.subcore_barrier()

    zero_acc()
    pltpu.sync_copy(dudi.at[1, pl.ds(s * RPT, RPT)], ddb)
    def s1_chunk(c, carry):
        rowbase = s * RPT + c * RC
        pltpu.sync_copy(e_ref.at[k].at[pl.ds(rowbase, RC)], rbuf)
        def scale_g(g, carry2):
            dvec = ddb[pl.ds(c * RC + g * 16, 16)]
            for l in range(16):
                r = g * 16 + l
                d = dvec[l]
                for j in range(4):
                    sl = pl.ds(j * 16, 16)
                    rbuf[r, sl] = rbuf[r, sl] * d
            return carry2
        lax.fori_loop(0, RC // 16, scale_g, None)
        pltpu.sync_copy(rbuf, s1_ref.at[k].at[pl.ds(rowbase, RC)])
        return carry
    lax.fori_loop(0, NRC, s1_chunk, None)
    plsc.subcore_barrier()

    def edge_pass(src_hbm, src_sel, dst_sel):
        src = src_hbm.at[k]
        def group(g, carry):
            pltpu.sync_copy(eidx_ref.at[s, g], eib)
            copies = []
            for j in range(G):
                copies.append(pltpu.async_copy(
                    src.at[eib.at[src_sel, j]], gbufs[j % 2], sems[j % 2]))
                if j > 0:
                    copies[j - 1].wait()
                    pltpu.sync_copy(gbufs[(j - 1) % 2],
                                    acc.at[eib.at[dst_sel, j - 1]], add=True)
            copies[G - 1].wait()
            pltpu.sync_copy(gbufs[(G - 1) % 2],
                            acc.at[eib.at[dst_sel, G - 1]], add=True)
            return carry
        lax.fori_loop(0, NG, group, None)
        plsc.subcore_barrier()

    def epilogue(dd_half, out_ref, scaled_ref, mode):
        pltpu.sync_copy(dudi.at[dd_half, pl.ds(s * RPT, RPT)], ddb)
        def chunk(c, carry):
            rowbase = s * RPT + c * RC
            pltpu.sync_copy(acc.at[pl.ds(rowbase, RC)], rbuf)
            if mode == "u2":
                pltpu.sync_copy(u0_ref.at[k].at[pl.ds(rowbase, RC)], sbuf)
            def row_g(g, carry2):
                dvec = ddb[pl.ds(c * RC + g * 16, 16)]
                for l in range(16):
                    r = g * 16 + l
                    d = dvec[l]
                    for j in range(4):
                        sl = pl.ds(j * 16, 16)
                        y = _tanh(rbuf[r, sl] * d)
                        if mode == "u0":
                            rbuf[r, sl] = y
                            sbuf[r, sl] = y * d
                        elif mode == "i1":
                            rbuf[r, sl] = 2.0 * y
                            sbuf[r, sl] = y * d
                        else:
                            rbuf[r, sl] = y + sbuf[r, sl]
                return carry2
            lax.fori_loop(0, RC // 16, row_g, None)
            pltpu.sync_copy(rbuf, out_ref.at[k].at[pl.ds(rowbase, RC)])
            if scaled_ref is not None:
                pltpu.sync_copy(sbuf, scaled_ref.at[k].at[pl.ds(rowbase, RC)])
            return carry
        lax.fori_loop(0, NRC, chunk, None)
        plsc.subcore_barrier()

    edge_pass(s1_ref, 1, 0)
    epilogue(0, u0_ref, s2_ref, "u0")
    zero_acc()
    plsc.subcore_barrier()
    edge_pass(s2_ref, 0, 1)
    epilogue(1, items_ref, s3_ref, "i1")
    zero_acc()
    plsc.subcore_barrier()
    edge_pass(s3_ref, 1, 0)
    epilogue(0, users_ref, None, "u2")


def kernel(item_emb, inter_rows, inter_cols):
    z = jnp.zeros((HR - N, 64), jnp.float32)
    e_pad = jnp.stack([
        jnp.concatenate([item_emb[:, :64], z], axis=0),
        jnp.concatenate([item_emb[:, 64:], z], axis=0),
    ])
    pad = jnp.full((NNZ_PAD - NNZ,), N, jnp.int32)
    rows_p = jnp.concatenate([inter_rows, pad]).reshape(16, NG, G, CK)
    cols_p = jnp.concatenate([inter_cols, pad]).reshape(16, NG, G, CK)
    eidx = jnp.stack([rows_p, cols_p], axis=2)
    users_r, items_r, _, _, _, _ = _egcf(e_pad, eidx)
    users = jnp.concatenate([users_r[0, :N], users_r[1, :N]], axis=1)
    items = jnp.concatenate([items_r[0, :N], items_r[1, :N]], axis=1)
    return users, items


# speedup vs baseline: 14.8316x; 1.1510x over previous
import functools

import jax
import jax.numpy as jnp
from jax import lax
from jax.experimental import pallas as pl
from jax.experimental.pallas import tpu as pltpu
from jax.experimental.pallas import tpu_sc as plsc

N = 25000
HR = 25088
NNZ = 400000
NNZ_PAD = 401408
EPT = NNZ_PAD // 16
CK = 128
G = 7
NG = EPT // (G * CK)
RPT = HR // 16
RC = 32
NRC = RPT // RC

_MAGIC = 0x5F3759DF


def _tanh(x):
    t = jnp.exp(2.0 * x)
    return 1.0 - 2.0 / (t + 1.0)


def _rsqrt_pos(x):
    xm = jnp.maximum(x, 1.0)
    ib = lax.bitcast_convert_type(xm, jnp.int32)
    ih = jnp.int32(_MAGIC) - lax.shift_right_logical(
        ib, jnp.full((16,), 1, jnp.int32))
    y = lax.bitcast_convert_type(ih, jnp.float32)
    y = y * (1.5 - 0.5 * xm * y * y)
    y = y * (1.5 - 0.5 * xm * y * y)
    y = y * (1.5 - 0.5 * xm * y * y)
    return jnp.where(x > 0.0, y, 0.0)


_mesh = plsc.VectorSubcoreMesh(core_axis_name="c", subcore_axis_name="s")


@functools.partial(
    pl.kernel,
    out_type=[jax.ShapeDtypeStruct((2, HR, 64), jnp.float32)
              for _ in range(6)],
    mesh=_mesh,
    compiler_params=pltpu.CompilerParams(use_tc_tiling_on_sc=False),
    scratch_types=[
        pltpu.VMEM_SHARED((HR, 64), jnp.float32),
        pltpu.VMEM_SHARED((2, HR), jnp.float32),
        pltpu.VMEM((2, G, CK), jnp.int32),
        pltpu.VMEM((CK, 64), jnp.float32),
        pltpu.VMEM((CK, 64), jnp.float32),
        pltpu.VMEM((RC, 64), jnp.float32),
        pltpu.VMEM((RC, 64), jnp.float32),
        pltpu.VMEM((RPT,), jnp.float32),
        pltpu.VMEM((CK,), jnp.float32),
        pltpu.SemaphoreType.DMA,
        pltpu.SemaphoreType.DMA,
    ],
)
def _egcf(e_ref, eidx_ref,
          users_ref, items_ref, u0_ref, s1_ref, s2_ref, s3_ref,
          acc, dudi, eib, gb0, gb1, rbuf, sbuf, ddb, onesb, sem0, sem1):
    k = lax.axis_index("c")
    s = lax.axis_index("s")
    gbufs = (gb0, gb1)
    sems = (sem0, sem1)

    def fill(ref, words, value):
        def body(i, carry):
            ref[pl.ds(i * 16, 16)] = jnp.full((16,), value, jnp.float32)
            return carry
        lax.fori_loop(0, words // 16, body, None)

    def fill2d(ref, rows, value, width=64):
        def body(r, carry):
            for j in range(width // 16):
                ref[r, pl.ds(j * 16, 16)] = jnp.full((16,), value, jnp.float32)
            return carry
        lax.fori_loop(0, rows, body, None)

    def for_dense_chunks(f):
        def body(c, carry):
            f(gb0, gb1, s * RPT + c * 128, 128)
            return carry
        lax.fori_loop(0, 12, body, None)
        f(rbuf, sbuf, s * RPT + 12 * 128, RC)

    def zero_acc():
        fill2d(gb0, 128, 0.0)
        fill2d(rbuf, RC, 0.0)
        def zc(bufR, bufS, base, rc):
            pltpu.sync_copy(bufR, acc.at[pl.ds(base, rc)])
        for_dense_chunks(zc)

    fill(ddb, RPT, 0.0)
    for h in range(2):
        pltpu.sync_copy(ddb, dudi.at[h, pl.ds(s * RPT, RPT)])
    plsc.subcore_barrier()

    fill(onesb, CK, 1.0)
    def deg_group(g, carry):
        pltpu.sync_copy(eidx_ref.at[s, g], eib)
        for j in range(G):
            pltpu.sync_copy(onesb, dudi.at[0].at[eib.at[0, j]], add=True)
            pltpu.sync_copy(onesb, dudi.at[1].at[eib.at[1, j]], add=True)
        return carry
    lax.fori_loop(0, NG, deg_group, None)
    plsc.subcore_barrier()

    for h in range(2):
        pltpu.sync_copy(dudi.at[h, pl.ds(s * RPT, RPT)], ddb)
        def rs_body(i, carry):
            sl = pl.ds(i * 16, 16)
            ddb[sl] = _rsqrt_pos(ddb[sl])
            return carry
        lax.fori_loop(0, RPT // 16, rs_body, None)
        pltpu.sync_copy(ddb, dudi.at[h, pl.ds(s * RPT, RPT)])
    plsc.subcore_barrier()

    zero_acc()
    pltpu.sync_copy(dudi.at[1, pl.ds(s * RPT, RPT)], ddb)
    def s1_chunk(bufR, bufS, base, rc):
        pltpu.sync_copy(e_ref.at[k].at[pl.ds(base, rc)], bufR)
        def scale_g(g, carry2):
            dvec = ddb[pl.ds(base - s * RPT + g * 16, 16)]
            for l in range(16):
                r = g * 16 + l
                d = dvec[l]
                for j in range(4):
                    sl = pl.ds(j * 16, 16)
                    bufR[r, sl] = bufR[r, sl] * d
            return carry2
        lax.fori_loop(0, rc // 16, scale_g, None)
        pltpu.sync_copy(bufR, s1_ref.at[k].at[pl.ds(base, rc)])
    for_dense_chunks(s1_chunk)
    plsc.subcore_barrier()

    def edge_pass(src_hbm, src_sel, dst_sel):
        src = src_hbm.at[k]
        def group(g, carry):
            pltpu.sync_copy(eidx_ref.at[s, g], eib)
            copies = []
            for j in range(G):
                copies.append(pltpu.async_copy(
                    src.at[eib.at[src_sel, j]], gbufs[j % 2], sems[j % 2]))
                if j > 0:
                    copies[j - 1].wait()
                    pltpu.sync_copy(gbufs[(j - 1) % 2],
                                    acc.at[eib.at[dst_sel, j - 1]], add=True)
            copies[G - 1].wait()
            pltpu.sync_copy(gbufs[(G - 1) % 2],
                            acc.at[eib.at[dst_sel, G - 1]], add=True)
            return carry
        lax.fori_loop(0, NG, group, None)
        plsc.subcore_barrier()

    def epilogue(dd_half, out_ref, scaled_ref, mode):
        pltpu.sync_copy(dudi.at[dd_half, pl.ds(s * RPT, RPT)], ddb)
        def chunk(bufR, bufS, base, rc):
            pltpu.sync_copy(acc.at[pl.ds(base, rc)], bufR)
            if mode == "u2":
                pltpu.sync_copy(u0_ref.at[k].at[pl.ds(base, rc)], bufS)
            def row_g(g, carry2):
                dvec = ddb[pl.ds(base - s * RPT + g * 16, 16)]
                for l in range(16):
                    r = g * 16 + l
                    d = dvec[l]
                    for j in range(4):
                        sl = pl.ds(j * 16, 16)
                        y = _tanh(bufR[r, sl] * d)
                        if mode == "u0":
                            bufR[r, sl] = y
                            bufS[r, sl] = y * d
                        elif mode == "i1":
                            bufR[r, sl] = 2.0 * y
                            bufS[r, sl] = y * d
                        else:
                            bufR[r, sl] = y + bufS[r, sl]
                return carry2
            lax.fori_loop(0, rc // 16, row_g, None)
            pltpu.sync_copy(bufR, out_ref.at[k].at[pl.ds(base, rc)])
            if scaled_ref is not None:
                pltpu.sync_copy(bufS, scaled_ref.at[k].at[pl.ds(base, rc)])
        for_dense_chunks(chunk)
        plsc.subcore_barrier()

    edge_pass(s1_ref, 1, 0)
    epilogue(0, u0_ref, s2_ref, "u0")
    zero_acc()
    plsc.subcore_barrier()
    edge_pass(s2_ref, 0, 1)
    epilogue(1, items_ref, s3_ref, "i1")
    zero_acc()
    plsc.subcore_barrier()
    edge_pass(s3_ref, 1, 0)
    epilogue(0, users_ref, None, "u2")


def kernel(item_emb, inter_rows, inter_cols):
    z = jnp.zeros((HR - N, 64), jnp.float32)
    e_pad = jnp.stack([
        jnp.concatenate([item_emb[:, :64], z], axis=0),
        jnp.concatenate([item_emb[:, 64:], z], axis=0),
    ])
    pad = jnp.full((NNZ_PAD - NNZ,), N, jnp.int32)
    rows_p = jnp.concatenate([inter_rows, pad]).reshape(16, NG, G, CK)
    cols_p = jnp.concatenate([inter_cols, pad]).reshape(16, NG, G, CK)
    eidx = jnp.stack([rows_p, cols_p], axis=2)
    users_r, items_r, _, _, _, _ = _egcf(e_pad, eidx)
    users = jnp.concatenate([users_r[0, :N], users_r[1, :N]], axis=1)
    items = jnp.concatenate([items_r[0, :N], items_r[1, :N]], axis=1)
    return users, items


# speedup vs baseline: 15.2079x; 1.0254x over previous
import functools

import jax
import jax.numpy as jnp
from jax import lax
from jax.experimental import pallas as pl
from jax.experimental.pallas import tpu as pltpu
from jax.experimental.pallas import tpu_sc as plsc

N = 25000
HR = 25088
NNZ = 400000
NNZ_PAD = 401408
EPT = NNZ_PAD // 16
CK = 128
G = 7
NG = EPT // (G * CK)
RPT = HR // 16
RC = 32
NRC = RPT // RC

_MAGIC = 0x5F3759DF


def _tanh(x):
    t = jnp.exp(2.0 * x)
    return 1.0 - 2.0 / (t + 1.0)


def _rsqrt_pos(x):
    xm = jnp.maximum(x, 1.0)
    ib = lax.bitcast_convert_type(xm, jnp.int32)
    ih = jnp.int32(_MAGIC) - lax.shift_right_logical(
        ib, jnp.full((16,), 1, jnp.int32))
    y = lax.bitcast_convert_type(ih, jnp.float32)
    y = y * (1.5 - 0.5 * xm * y * y)
    y = y * (1.5 - 0.5 * xm * y * y)
    y = y * (1.5 - 0.5 * xm * y * y)
    return jnp.where(x > 0.0, y, 0.0)


_mesh = plsc.VectorSubcoreMesh(core_axis_name="c", subcore_axis_name="s")


@functools.partial(
    pl.kernel,
    out_type=[jax.ShapeDtypeStruct((2, HR, 64), jnp.float32)
              for _ in range(6)],
    mesh=_mesh,
    compiler_params=pltpu.CompilerParams(use_tc_tiling_on_sc=False),
    scratch_types=[
        pltpu.VMEM_SHARED((HR, 64), jnp.float32),
        pltpu.VMEM_SHARED((2, HR), jnp.float32),
        pltpu.VMEM((2, G, CK), jnp.int32),
        pltpu.VMEM((CK, 64), jnp.float32),
        pltpu.VMEM((CK, 64), jnp.float32),
        pltpu.VMEM((RC, 64), jnp.float32),
        pltpu.VMEM((RC, 64), jnp.float32),
        pltpu.VMEM((RPT,), jnp.float32),
        pltpu.VMEM((CK,), jnp.float32),
        pltpu.SemaphoreType.DMA,
        pltpu.SemaphoreType.DMA,
        pltpu.SemaphoreType.DMA,
        pltpu.SemaphoreType.DMA,
    ],
)
def _egcf(e_ref, eidx_ref,
          users_ref, items_ref, u0_ref, s1_ref, s2_ref, s3_ref,
          acc, dudi, eib, gb0, gb1, rbuf, sbuf, ddb, onesb,
          sem0, sem1, sem2, sem3):
    k = lax.axis_index("c")
    s = lax.axis_index("s")
    gbufs = (gb0, gb1)
    sems = (sem0, sem1)
    ssems = (sem2, sem3)

    def fill(ref, words, value):
        def body(i, carry):
            ref[pl.ds(i * 16, 16)] = jnp.full((16,), value, jnp.float32)
            return carry
        lax.fori_loop(0, words // 16, body, None)

    def fill2d(ref, rows, value, width=64):
        def body(r, carry):
            for j in range(width // 16):
                ref[r, pl.ds(j * 16, 16)] = jnp.full((16,), value, jnp.float32)
            return carry
        lax.fori_loop(0, rows, body, None)

    def for_dense_chunks(f):
        def body(c, carry):
            f(gb0, gb1, s * RPT + c * 128, 128)
            return carry
        lax.fori_loop(0, 12, body, None)
        f(rbuf, sbuf, s * RPT + 12 * 128, RC)

    def zero_acc():
        fill2d(gb0, 128, 0.0)
        fill2d(rbuf, RC, 0.0)
        def zc(bufR, bufS, base, rc):
            pltpu.sync_copy(bufR, acc.at[pl.ds(base, rc)])
        for_dense_chunks(zc)

    fill(ddb, RPT, 0.0)
    for h in range(2):
        pltpu.sync_copy(ddb, dudi.at[h, pl.ds(s * RPT, RPT)])
    plsc.subcore_barrier()

    fill(onesb, CK, 1.0)
    def deg_group(g, carry):
        pltpu.sync_copy(eidx_ref.at[s, g], eib)
        copies = []
        for j in range(G):
            copies.append(pltpu.async_copy(
                onesb, dudi.at[0].at[eib.at[0, j]], sem0, add=True))
            copies.append(pltpu.async_copy(
                onesb, dudi.at[1].at[eib.at[1, j]], sem1, add=True))
        for cp in copies:
            cp.wait()
        return carry
    lax.fori_loop(0, NG, deg_group, None)
    plsc.subcore_barrier()

    for h in range(2):
        pltpu.sync_copy(dudi.at[h, pl.ds(s * RPT, RPT)], ddb)
        def rs_body(i, carry):
            sl = pl.ds(i * 16, 16)
            ddb[sl] = _rsqrt_pos(ddb[sl])
            return carry
        lax.fori_loop(0, RPT // 16, rs_body, None)
        pltpu.sync_copy(ddb, dudi.at[h, pl.ds(s * RPT, RPT)])
    plsc.subcore_barrier()

    zero_acc()
    pltpu.sync_copy(dudi.at[1, pl.ds(s * RPT, RPT)], ddb)
    def s1_chunk(bufR, bufS, base, rc):
        pltpu.sync_copy(e_ref.at[k].at[pl.ds(base, rc)], bufR)
        def scale_g(g, carry2):
            dvec = ddb[pl.ds(base - s * RPT + g * 16, 16)]
            for l in range(16):
                r = g * 16 + l
                d = dvec[l]
                for j in range(4):
                    sl = pl.ds(j * 16, 16)
                    bufR[r, sl] = bufR[r, sl] * d
            return carry2
        lax.fori_loop(0, rc // 16, scale_g, None)
        pltpu.sync_copy(bufR, s1_ref.at[k].at[pl.ds(base, rc)])
    for_dense_chunks(s1_chunk)
    plsc.subcore_barrier()

    def edge_pass(src_hbm, src_sel, dst_sel):
        src = src_hbm.at[k]
        def gather(j):
            return pltpu.async_copy(
                src.at[eib.at[src_sel, j]], gbufs[j % 2], sems[j % 2])
        def scatter(j):
            return pltpu.async_copy(
                gbufs[j % 2], acc.at[eib.at[dst_sel, j]], ssems[j % 2],
                add=True)
        def group(g, carry):
            pltpu.sync_copy(eidx_ref.at[s, g], eib)
            gc = [None] * G
            sc = [None] * G
            gc[0] = gather(0)
            for j in range(G):
                if j + 1 < G:
                    if j >= 1:
                        sc[j - 1].wait()
                    gc[j + 1] = gather(j + 1)
                gc[j].wait()
                sc[j] = scatter(j)
            sc[G - 2].wait()
            sc[G - 1].wait()
            return carry
        lax.fori_loop(0, NG, group, None)
        plsc.subcore_barrier()

    def epilogue(dd_half, out_ref, scaled_ref, mode):
        pltpu.sync_copy(dudi.at[dd_half, pl.ds(s * RPT, RPT)], ddb)
        def chunk(bufR, bufS, base, rc):
            pltpu.sync_copy(acc.at[pl.ds(base, rc)], bufR)
            if mode == "u2":
                pltpu.sync_copy(u0_ref.at[k].at[pl.ds(base, rc)], bufS)
            def row_g(g, carry2):
                dvec = ddb[pl.ds(base - s * RPT + g * 16, 16)]
                for l in range(16):
                    r = g * 16 + l
                    d = dvec[l]
                    for j in range(4):
                        sl = pl.ds(j * 16, 16)
                        y = _tanh(bufR[r, sl] * d)
                        if mode == "u0":
                            bufR[r, sl] = y
                            bufS[r, sl] = y * d
                        elif mode == "i1":
                            bufR[r, sl] = 2.0 * y
                            bufS[r, sl] = y * d
                        else:
                            bufR[r, sl] = y + bufS[r, sl]
                return carry2
            lax.fori_loop(0, rc // 16, row_g, None)
            pltpu.sync_copy(bufR, out_ref.at[k].at[pl.ds(base, rc)])
            if scaled_ref is not None:
                pltpu.sync_copy(bufS, scaled_ref.at[k].at[pl.ds(base, rc)])
        for_dense_chunks(chunk)
        plsc.subcore_barrier()

    edge_pass(s1_ref, 1, 0)
    epilogue(0, u0_ref, s2_ref, "u0")
    zero_acc()
    plsc.subcore_barrier()
    edge_pass(s2_ref, 0, 1)
    epilogue(1, items_ref, s3_ref, "i1")
    zero_acc()
    plsc.subcore_barrier()
    edge_pass(s3_ref, 1, 0)
    epilogue(0, users_ref, None, "u2")


def kernel(item_emb, inter_rows, inter_cols):
    z = jnp.zeros((HR - N, 64), jnp.float32)
    e_pad = jnp.stack([
        jnp.concatenate([item_emb[:, :64], z], axis=0),
        jnp.concatenate([item_emb[:, 64:], z], axis=0),
    ])
    pad = jnp.full((NNZ_PAD - NNZ,), N, jnp.int32)
    rows_p = jnp.concatenate([inter_rows, pad]).reshape(16, NG, G, CK)
    cols_p = jnp.concatenate([inter_cols, pad]).reshape(16, NG, G, CK)
    eidx = jnp.stack([rows_p, cols_p], axis=2)
    users_r, items_r, _, _, _, _ = _egcf(e_pad, eidx)
    users = jnp.concatenate([users_r[0, :N], users_r[1, :N]], axis=1)
    items = jnp.concatenate([items_r[0, :N], items_r[1, :N]], axis=1)
    return users, items


# speedup vs baseline: 16.3850x; 1.0774x over previous
import functools

import jax
import jax.numpy as jnp
from jax import lax
from jax.experimental import pallas as pl
from jax.experimental.pallas import tpu as pltpu
from jax.experimental.pallas import tpu_sc as plsc

N = 25000
HR = 25088
NNZ = 400000
NNZ_PAD = 401408
EPT = NNZ_PAD // 16
CK = 128
G = 14
NG = EPT // (G * CK)
RPT = HR // 16
RC = 32
NRC = RPT // RC

_MAGIC = 0x5F3759DF


def _tanh(x):
    t = jnp.exp(2.0 * x)
    return 1.0 - 2.0 / (t + 1.0)


def _rsqrt_pos(x):
    xm = jnp.maximum(x, 1.0)
    ib = lax.bitcast_convert_type(xm, jnp.int32)
    ih = jnp.int32(_MAGIC) - lax.shift_right_logical(
        ib, jnp.full((16,), 1, jnp.int32))
    y = lax.bitcast_convert_type(ih, jnp.float32)
    y = y * (1.5 - 0.5 * xm * y * y)
    y = y * (1.5 - 0.5 * xm * y * y)
    y = y * (1.5 - 0.5 * xm * y * y)
    return jnp.where(x > 0.0, y, 0.0)


_mesh = plsc.VectorSubcoreMesh(core_axis_name="c", subcore_axis_name="s")


@functools.partial(
    pl.kernel,
    out_type=[jax.ShapeDtypeStruct((2, HR, 64), jnp.float32)
              for _ in range(6)],
    mesh=_mesh,
    compiler_params=pltpu.CompilerParams(use_tc_tiling_on_sc=False),
    scratch_types=[
        pltpu.VMEM_SHARED((HR, 64), jnp.float32),
        pltpu.VMEM_SHARED((2, HR), jnp.float32),
        pltpu.VMEM((2, G, CK), jnp.int32),
        pltpu.VMEM((CK, 64), jnp.float32),
        pltpu.VMEM((CK, 64), jnp.float32),
        pltpu.VMEM((RC, 64), jnp.float32),
        pltpu.VMEM((RC, 64), jnp.float32),
        pltpu.VMEM((RPT,), jnp.float32),
        pltpu.VMEM((CK,), jnp.float32),
        pltpu.SemaphoreType.DMA,
        pltpu.SemaphoreType.DMA,
        pltpu.SemaphoreType.DMA,
        pltpu.SemaphoreType.DMA,
    ],
)
def _egcf(e_ref, eidx_ref,
          users_ref, items_ref, u0_ref, s1_ref, s2_ref, s3_ref,
          acc, dudi, eib, gb0, gb1, rbuf, sbuf, ddb, onesb,
          sem0, sem1, sem2, sem3):
    k = lax.axis_index("c")
    s = lax.axis_index("s")
    gbufs = (gb0, gb1)
    sems = (sem0, sem1)
    ssems = (sem2, sem3)

    def fill(ref, words, value):
        def body(i, carry):
            ref[pl.ds(i * 16, 16)] = jnp.full((16,), value, jnp.float32)
            return carry
        lax.fori_loop(0, words // 16, body, None)

    def fill2d(ref, rows, value, width=64):
        def body(r, carry):
            for j in range(width // 16):
                ref[r, pl.ds(j * 16, 16)] = jnp.full((16,), value, jnp.float32)
            return carry
        lax.fori_loop(0, rows, body, None)

    def for_dense_chunks(f):
        def body(c, carry):
            f(gb0, gb1, s * RPT + c * 128, 128)
            return carry
        lax.fori_loop(0, 12, body, None)
        f(rbuf, sbuf, s * RPT + 12 * 128, RC)

    def zero_acc():
        fill2d(gb0, 128, 0.0)
        fill2d(rbuf, RC, 0.0)
        def zc(bufR, bufS, base, rc):
            pltpu.sync_copy(bufR, acc.at[pl.ds(base, rc)])
        for_dense_chunks(zc)

    fill(ddb, RPT, 0.0)
    for h in range(2):
        pltpu.sync_copy(ddb, dudi.at[h, pl.ds(s * RPT, RPT)])
    plsc.subcore_barrier()

    fill(onesb, CK, 1.0)
    def deg_group(g, carry):
        pltpu.sync_copy(eidx_ref.at[s, g], eib)
        copies = []
        for j in range(G):
            copies.append(pltpu.async_copy(
                onesb, dudi.at[0].at[eib.at[0, j]], sem0, add=True))
            copies.append(pltpu.async_copy(
                onesb, dudi.at[1].at[eib.at[1, j]], sem1, add=True))
        for cp in copies:
            cp.wait()
        return carry
    lax.fori_loop(0, NG, deg_group, None)
    plsc.subcore_barrier()

    for h in range(2):
        pltpu.sync_copy(dudi.at[h, pl.ds(s * RPT, RPT)], ddb)
        def rs_body(i, carry):
            sl = pl.ds(i * 16, 16)
            ddb[sl] = _rsqrt_pos(ddb[sl])
            return carry
        lax.fori_loop(0, RPT // 16, rs_body, None)
        pltpu.sync_copy(ddb, dudi.at[h, pl.ds(s * RPT, RPT)])
    plsc.subcore_barrier()

    zero_acc()
    pltpu.sync_copy(dudi.at[1, pl.ds(s * RPT, RPT)], ddb)
    def s1_chunk(bufR, bufS, base, rc):
        pltpu.sync_copy(e_ref.at[k].at[pl.ds(base, rc)], bufR)
        def scale_g(g, carry2):
            dvec = ddb[pl.ds(base - s * RPT + g * 16, 16)]
            for l in range(16):
                r = g * 16 + l
                d = dvec[l]
                for j in range(4):
                    sl = pl.ds(j * 16, 16)
                    bufR[r, sl] = bufR[r, sl] * d
            return carry2
        lax.fori_loop(0, rc // 16, scale_g, None)
        pltpu.sync_copy(bufR, s1_ref.at[k].at[pl.ds(base, rc)])
    for_dense_chunks(s1_chunk)
    plsc.subcore_barrier()

    def edge_pass(src_hbm, src_sel, dst_sel):
        src = src_hbm.at[k]
        def gather(j):
            return pltpu.async_copy(
                src.at[eib.at[src_sel, j]], gbufs[j % 2], sems[j % 2])
        def scatter(j):
            return pltpu.async_copy(
                gbufs[j % 2], acc.at[eib.at[dst_sel, j]], ssems[j % 2],
                add=True)
        def group(g, carry):
            pltpu.sync_copy(eidx_ref.at[s, g], eib)
            gc = [None] * G
            sc = [None] * G
            gc[0] = gather(0)
            for j in range(G):
                if j + 1 < G:
                    if j >= 1:
                        sc[j - 1].wait()
                    gc[j + 1] = gather(j + 1)
                gc[j].wait()
                sc[j] = scatter(j)
            sc[G - 2].wait()
            sc[G - 1].wait()
            return carry
        lax.fori_loop(0, NG, group, None)
        plsc.subcore_barrier()

    def epilogue(dd_half, out_ref, scaled_ref, mode):
        pltpu.sync_copy(dudi.at[dd_half, pl.ds(s * RPT, RPT)], ddb)
        def chunk(bufR, bufS, base, rc):
            pltpu.sync_copy(acc.at[pl.ds(base, rc)], bufR)
            if mode == "u2":
                pltpu.sync_copy(u0_ref.at[k].at[pl.ds(base, rc)], bufS)
            def row_g(g, carry2):
                dvec = ddb[pl.ds(base - s * RPT + g * 16, 16)]
                for l in range(16):
                    r = g * 16 + l
                    d = dvec[l]
                    for j in range(4):
                        sl = pl.ds(j * 16, 16)
                        y = _tanh(bufR[r, sl] * d)
                        if mode == "u0":
                            bufR[r, sl] = y
                            bufS[r, sl] = y * d
                        elif mode == "i1":
                            bufR[r, sl] = 2.0 * y
                            bufS[r, sl] = y * d
                        else:
                            bufR[r, sl] = y + bufS[r, sl]
                return carry2
            lax.fori_loop(0, rc // 16, row_g, None)
            pltpu.sync_copy(bufR, out_ref.at[k].at[pl.ds(base, rc)])
            if scaled_ref is not None:
                pltpu.sync_copy(bufS, scaled_ref.at[k].at[pl.ds(base, rc)])
        for_dense_chunks(chunk)
        plsc.subcore_barrier()

    edge_pass(s1_ref, 1, 0)
    epilogue(0, u0_ref, s2_ref, "u0")
    zero_acc()
    plsc.subcore_barrier()
    edge_pass(s2_ref, 0, 1)
    epilogue(1, items_ref, s3_ref, "i1")
    zero_acc()
    plsc.subcore_barrier()
    edge_pass(s3_ref, 1, 0)
    epilogue(0, users_ref, None, "u2")


def kernel(item_emb, inter_rows, inter_cols):
    z = jnp.zeros((HR - N, 64), jnp.float32)
    e_pad = jnp.stack([
        jnp.concatenate([item_emb[:, :64], z], axis=0),
        jnp.concatenate([item_emb[:, 64:], z], axis=0),
    ])
    pad = jnp.full((NNZ_PAD - NNZ,), N, jnp.int32)
    rows_p = jnp.concatenate([inter_rows, pad]).reshape(16, NG, G, CK)
    cols_p = jnp.concatenate([inter_cols, pad]).reshape(16, NG, G, CK)
    eidx = jnp.stack([rows_p, cols_p], axis=2)
    users_r, items_r, _, _, _, _ = _egcf(e_pad, eidx)
    users = jnp.concatenate([users_r[0, :N], users_r[1, :N]], axis=1)
    items = jnp.concatenate([items_r[0, :N], items_r[1, :N]], axis=1)
    return users, items


# speedup vs baseline: 16.8643x; 1.0293x over previous
import functools

import jax
import jax.numpy as jnp
from jax import lax
from jax.experimental import pallas as pl
from jax.experimental.pallas import tpu as pltpu
from jax.experimental.pallas import tpu_sc as plsc

N = 25000
HR = 25088
NNZ = 400000
NNZ_PAD = 401408
EPT = NNZ_PAD // 16
CK = 128
G = 28
NG = EPT // (G * CK)
RPT = HR // 16
RC = 32
NRC = RPT // RC

_MAGIC = 0x5F3759DF


def _tanh(x):
    t = jnp.exp(2.0 * x)
    return 1.0 - 2.0 / (t + 1.0)


def _rsqrt_pos(x):
    xm = jnp.maximum(x, 1.0)
    ib = lax.bitcast_convert_type(xm, jnp.int32)
    ih = jnp.int32(_MAGIC) - lax.shift_right_logical(
        ib, jnp.full((16,), 1, jnp.int32))
    y = lax.bitcast_convert_type(ih, jnp.float32)
    y = y * (1.5 - 0.5 * xm * y * y)
    y = y * (1.5 - 0.5 * xm * y * y)
    y = y * (1.5 - 0.5 * xm * y * y)
    return jnp.where(x > 0.0, y, 0.0)


_mesh = plsc.VectorSubcoreMesh(core_axis_name="c", subcore_axis_name="s")


@functools.partial(
    pl.kernel,
    out_type=[jax.ShapeDtypeStruct((2, HR, 64), jnp.float32)
              for _ in range(6)],
    mesh=_mesh,
    compiler_params=pltpu.CompilerParams(use_tc_tiling_on_sc=False),
    scratch_types=[
        pltpu.VMEM_SHARED((HR, 64), jnp.float32),
        pltpu.VMEM_SHARED((2, HR), jnp.float32),
        pltpu.VMEM((2, G, CK), jnp.int32),
        pltpu.VMEM((CK, 64), jnp.float32),
        pltpu.VMEM((CK, 64), jnp.float32),
        pltpu.VMEM((RPT,), jnp.float32),
        pltpu.VMEM((CK,), jnp.float32),
        pltpu.SemaphoreType.DMA,
        pltpu.SemaphoreType.DMA,
        pltpu.SemaphoreType.DMA,
        pltpu.SemaphoreType.DMA,
    ],
)
def _egcf(e_ref, eidx_ref,
          users_ref, items_ref, u0_ref, s1_ref, s2_ref, s3_ref,
          acc, dudi, eib, gb0, gb1, ddb, onesb,
          sem0, sem1, sem2, sem3):
    k = lax.axis_index("c")
    s = lax.axis_index("s")
    gbufs = (gb0, gb1)
    sems = (sem0, sem1)
    ssems = (sem2, sem3)

    def fill(ref, words, value):
        def body(i, carry):
            ref[pl.ds(i * 16, 16)] = jnp.full((16,), value, jnp.float32)
            return carry
        lax.fori_loop(0, words // 16, body, None)

    def fill2d(ref, rows, value, width=64):
        def body(r, carry):
            for j in range(width // 16):
                ref[r, pl.ds(j * 16, 16)] = jnp.full((16,), value, jnp.float32)
            return carry
        lax.fori_loop(0, rows, body, None)

    def for_dense_chunks(f):
        def body(c, carry):
            f(s * RPT + c * 128, 128)
            return carry
        lax.fori_loop(0, 12, body, None)
        f(s * RPT + 12 * 128, RC)

    def zero_acc():
        fill2d(gb0, 128, 0.0)
        def zc(base, rc):
            pltpu.sync_copy(gb0.at[pl.ds(0, rc)], acc.at[pl.ds(base, rc)])
        for_dense_chunks(zc)

    fill(ddb, RPT, 0.0)
    for h in range(2):
        pltpu.sync_copy(ddb, dudi.at[h, pl.ds(s * RPT, RPT)])
    plsc.subcore_barrier()

    fill(onesb, CK, 1.0)
    def deg_group(g, carry):
        pltpu.sync_copy(eidx_ref.at[s, g], eib)
        copies = []
        for j in range(G):
            copies.append(pltpu.async_copy(
                onesb, dudi.at[0].at[eib.at[0, j]], sem0, add=True))
            copies.append(pltpu.async_copy(
                onesb, dudi.at[1].at[eib.at[1, j]], sem1, add=True))
        for cp in copies:
            cp.wait()
        return carry
    lax.fori_loop(0, NG, deg_group, None)
    plsc.subcore_barrier()

    for h in range(2):
        pltpu.sync_copy(dudi.at[h, pl.ds(s * RPT, RPT)], ddb)
        def rs_body(i, carry):
            sl = pl.ds(i * 16, 16)
            ddb[sl] = _rsqrt_pos(ddb[sl])
            return carry
        lax.fori_loop(0, RPT // 16, rs_body, None)
        pltpu.sync_copy(ddb, dudi.at[h, pl.ds(s * RPT, RPT)])
    plsc.subcore_barrier()

    zero_acc()
    pltpu.sync_copy(dudi.at[1, pl.ds(s * RPT, RPT)], ddb)
    def s1_chunk(base, rc):
        pltpu.sync_copy(e_ref.at[k].at[pl.ds(base, rc)], gb0.at[pl.ds(0, rc)])
        def scale_g(g, carry2):
            dvec = ddb[pl.ds(base - s * RPT + g * 16, 16)]
            for l in range(16):
                r = g * 16 + l
                d = dvec[l]
                for j in range(4):
                    sl = pl.ds(j * 16, 16)
                    gb0[r, sl] = gb0[r, sl] * d
            return carry2
        lax.fori_loop(0, rc // 16, scale_g, None)
        pltpu.sync_copy(gb0.at[pl.ds(0, rc)], s1_ref.at[k].at[pl.ds(base, rc)])
    for_dense_chunks(s1_chunk)
    plsc.subcore_barrier()

    def edge_pass(src_hbm, src_sel, dst_sel):
        src = src_hbm.at[k]
        def gather(j):
            return pltpu.async_copy(
                src.at[eib.at[src_sel, j]], gbufs[j % 2], sems[j % 2])
        def scatter(j):
            return pltpu.async_copy(
                gbufs[j % 2], acc.at[eib.at[dst_sel, j]], ssems[j % 2],
                add=True)
        def group(g, carry):
            pltpu.sync_copy(eidx_ref.at[s, g], eib)
            gc = [None] * G
            sc = [None] * G
            gc[0] = gather(0)
            for j in range(G):
                if j + 1 < G:
                    if j >= 1:
                        sc[j - 1].wait()
                    gc[j + 1] = gather(j + 1)
                gc[j].wait()
                sc[j] = scatter(j)
            sc[G - 2].wait()
            sc[G - 1].wait()
            return carry
        lax.fori_loop(0, NG, group, None)
        plsc.subcore_barrier()

    def epilogue(dd_half, out_ref, scaled_ref, mode):
        pltpu.sync_copy(dudi.at[dd_half, pl.ds(s * RPT, RPT)], ddb)
        def chunk(base, rc):
            pltpu.sync_copy(acc.at[pl.ds(base, rc)], gb0.at[pl.ds(0, rc)])
            if mode == "u2":
                pltpu.sync_copy(u0_ref.at[k].at[pl.ds(base, rc)],
                                gb1.at[pl.ds(0, rc)])
            def row_g(g, carry2):
                dvec = ddb[pl.ds(base - s * RPT + g * 16, 16)]
                for l in range(16):
                    r = g * 16 + l
                    d = dvec[l]
                    for j in range(4):
                        sl = pl.ds(j * 16, 16)
                        y = _tanh(gb0[r, sl] * d)
                        if mode == "u0":
                            gb0[r, sl] = y
                            gb1[r, sl] = y * d
                        elif mode == "i1":
                            gb0[r, sl] = 2.0 * y
                            gb1[r, sl] = y * d
                        else:
                            gb0[r, sl] = y + gb1[r, sl]
                return carry2
            lax.fori_loop(0, rc // 16, row_g, None)
            pltpu.sync_copy(gb0.at[pl.ds(0, rc)],
                            out_ref.at[k].at[pl.ds(base, rc)])
            if scaled_ref is not None:
                pltpu.sync_copy(gb1.at[pl.ds(0, rc)],
                                scaled_ref.at[k].at[pl.ds(base, rc)])
        for_dense_chunks(chunk)
        plsc.subcore_barrier()

    edge_pass(s1_ref, 1, 0)
    epilogue(0, u0_ref, s2_ref, "u0")
    zero_acc()
    plsc.subcore_barrier()
    edge_pass(s2_ref, 0, 1)
    epilogue(1, items_ref, s3_ref, "i1")
    zero_acc()
    plsc.subcore_barrier()
    edge_pass(s3_ref, 1, 0)
    epilogue(0, users_ref, None, "u2")


def kernel(item_emb, inter_rows, inter_cols):
    z = jnp.zeros((HR - N, 64), jnp.float32)
    e_pad = jnp.stack([
        jnp.concatenate([item_emb[:, :64], z], axis=0),
        jnp.concatenate([item_emb[:, 64:], z], axis=0),
    ])
    pad = jnp.full((NNZ_PAD - NNZ,), N, jnp.int32)
    rows_p = jnp.concatenate([inter_rows, pad]).reshape(16, NG, G, CK)
    cols_p = jnp.concatenate([inter_cols, pad]).reshape(16, NG, G, CK)
    eidx = jnp.stack([rows_p, cols_p], axis=2)
    users_r, items_r, _, _, _, _ = _egcf(e_pad, eidx)
    users = jnp.concatenate([users_r[0, :N], users_r[1, :N]], axis=1)
    items = jnp.concatenate([items_r[0, :N], items_r[1, :N]], axis=1)
    return users, items
